# baseline (speedup 1.0000x reference)
import jax
import jax.numpy as jnp
from jax.experimental import pallas as pl

_NUM_SKILL = 100
_EMB_DIM = 16
_S_BLK = 2


def _concept_kernel(conc_ref, emb_bc_ref, out_ref):
    nb = conc_ref.shape[2]

    for si in range(_S_BLK):
        c0 = conc_ref[si, 0, :]
        c1 = conc_ref[si, 1, :]
        c2 = conc_ref[si, 2, :]
        c3 = conc_ref[si, 3, :]

        def body(n, _, c0=c0, c1=c1, c2=c2, c3=c3, si=si):
            cnt = (
                (c0 == n).astype(jnp.float32)
                + (c1 == n).astype(jnp.float32)
                + (c2 == n).astype(jnp.float32)
                + (c3 == n).astype(jnp.float32)
            )
            out_ref[si, n] = (
                jnp.broadcast_to(cnt[None, :], (_EMB_DIM, nb)) * emb_bc_ref[n]
            )
            return 0

        jax.lax.fori_loop(0, _NUM_SKILL, body, 0)


def kernel(concepts, emb_table_skill):
    b, s, k = concepts.shape
    conc_t = jnp.transpose(concepts.astype(jnp.int32), (1, 2, 0))
    emb_bc = jnp.broadcast_to(
        emb_table_skill[:_NUM_SKILL, :, None], (_NUM_SKILL, _EMB_DIM, b)
    )

    out_t = pl.pallas_call(
        _concept_kernel,
        grid=(s // _S_BLK,),
        in_specs=[
            pl.BlockSpec((_S_BLK, k, b), lambda i: (i, 0, 0)),
            pl.BlockSpec((_NUM_SKILL, _EMB_DIM, b), lambda i: (0, 0, 0)),
        ],
        out_specs=pl.BlockSpec((_S_BLK, _NUM_SKILL, _EMB_DIM, b), lambda i: (i, 0, 0, 0)),
        out_shape=jax.ShapeDtypeStruct((s, _NUM_SKILL, _EMB_DIM, b), jnp.float32),
    )(conc_t, emb_bc)
    return jnp.transpose(out_t, (3, 0, 1, 2))
